# baseline (speedup 1.0000x reference)
import functools

import jax
import jax.numpy as jnp
from jax import lax
from jax.experimental import pallas as pl
from jax.experimental.pallas import tpu as pltpu
from jax.experimental.pallas import tpu_sc as plsc

N_NODE = 10000
D_IN = 128
H = 64
E = 320000

NC = 2
NS = 16
TILES = NC * NS
CHUNK = 128
ROWS_PER_TILE = 80
N_ROWS = TILES * ROWS_PER_TILE
E_PAD = N_ROWS * CHUNK
DUMMY = N_NODE
ACC_ROWS = 10240
SEG_PER_TILE = ACC_ROWS // NS
CW = 16
NBUF = 4

BLK = 200


def _proj_body(u_ref, p_ref, wld, wls, wlp, wrd, wrs, wrp, bld, bls, blp,
               yd, ys, yp, ru, rp):
    u = u_ref[...]
    p = p_ref[...]
    f32 = jnp.float32
    yd[...] = jnp.dot(p, wld[...], preferred_element_type=f32)
    ys[...] = jnp.dot(u, wls[...], preferred_element_type=f32)
    yp[...] = jnp.dot(u, wlp[...], preferred_element_type=f32)
    ru[...] = (jnp.dot(u, wrd[...], preferred_element_type=f32)
               + 0.75 * jnp.dot(u, wrs[...], preferred_element_type=f32)
               + bld[...] + 0.75 * bls[...])
    rp[...] = jnp.dot(p, wrp[...], preferred_element_type=f32) + blp[...]


def _project(user_x, post_x, Wl_d, Wl_s, Wl_p, Wr_d, Wr_s, Wr_p, bl_d, bl_s, bl_p):
    n_blk = N_NODE // BLK
    row_spec = pl.BlockSpec((BLK, D_IN), lambda i: (i, 0))
    w_spec = pl.BlockSpec((D_IN, H), lambda i: (0, 0))
    b_spec = pl.BlockSpec((1, H), lambda i: (0, 0))
    out_spec = pl.BlockSpec((BLK, H), lambda i: (i, 0))
    out_shape = jax.ShapeDtypeStruct((N_NODE, H), jnp.float32)
    return pl.pallas_call(
        _proj_body,
        grid=(n_blk,),
        in_specs=[row_spec, row_spec] + [w_spec] * 6 + [b_spec] * 3,
        out_specs=[out_spec] * 5,
        out_shape=[out_shape] * 5,
    )(user_x, post_x, Wl_d, Wl_s, Wl_p, Wr_d, Wr_s, Wr_p,
      bl_d.reshape(1, H), bl_s.reshape(1, H), bl_p.reshape(1, H))


def _sc_body(yd_h, ys_h, yp_h, sd_h, dd_h, ss_h, ds_h, sp_h, dp_h,
             ones_h, zseg_h, zcnt_h,
             segd_o, segs_o, segp_o, cntd_o, cnts_o, cntp_o,
             src_v, dst_v, rows_v, ones_v,
             seg_sh, cnt_sh,
             g0, g1, g2, g3, s0, s1, s2, s3):
    cid = lax.axis_index("c")
    sid = lax.axis_index("s")
    tid = sid * NC + cid
    g_sems = (g0, g1, g2, g3)
    s_sems = (s0, s1, s2, s3)

    pltpu.sync_copy(ones_h, ones_v)

    my_acc = pl.ds(sid * SEG_PER_TILE, SEG_PER_TILE)

    def run_rel(src_h, dst_h, y_h, seg_o, cnt_o):
        pltpu.sync_copy(zseg_h, seg_sh.at[my_acc])
        pltpu.sync_copy(zcnt_h, cnt_sh.at[my_acc])
        row0 = tid * ROWS_PER_TILE
        pltpu.sync_copy(src_h.at[pl.ds(row0, ROWS_PER_TILE)], src_v)
        pltpu.sync_copy(dst_h.at[pl.ds(row0, ROWS_PER_TILE)], dst_v)
        plsc.subcore_barrier()

        def start_gather(row, b):
            pltpu.async_copy(y_h.at[src_v.at[row]], rows_v.at[b], g_sems[b])

        def wait_gather(row, b):
            pltpu.make_async_copy(y_h.at[src_v.at[row]], rows_v.at[b],
                                  g_sems[b]).wait()

        def start_scatter(row, b):
            pltpu.async_copy(rows_v.at[b], seg_sh.at[dst_v.at[row]],
                             s_sems[b], add=True)
            pltpu.async_copy(ones_v, cnt_sh.at[dst_v.at[row]],
                             s_sems[b], add=True)

        def wait_scatter(row, b):
            pltpu.make_async_copy(rows_v.at[b], seg_sh.at[dst_v.at[row]],
                                  s_sems[b]).wait()
            pltpu.make_async_copy(ones_v, cnt_sh.at[dst_v.at[row]],
                                  s_sems[b]).wait()

        start_gather(0, 0)
        start_gather(1, 1)

        def step(j4, _):
            j0 = j4 * NBUF
            for b in range(NBUF):
                j = j0 + b
                wait_gather(j, b)
                start_scatter(j, b)
                b2 = (b + 2) % NBUF

                @pl.when(j >= 2)
                def _():
                    wait_scatter(j - 2, b2)

                @pl.when(j + 2 < ROWS_PER_TILE)
                def _():
                    start_gather(j + 2, b2)
            return _

        lax.fori_loop(0, ROWS_PER_TILE // NBUF, step, None)
        wait_scatter(ROWS_PER_TILE - 2, (ROWS_PER_TILE - 2) % NBUF)
        wait_scatter(ROWS_PER_TILE - 1, (ROWS_PER_TILE - 1) % NBUF)
        plsc.subcore_barrier()
        pltpu.sync_copy(seg_sh.at[my_acc], seg_o.at[cid, my_acc])
        pltpu.sync_copy(cnt_sh.at[my_acc], cnt_o.at[cid, my_acc])

    run_rel(sd_h, dd_h, yd_h, segd_o, cntd_o)
    run_rel(ss_h, ds_h, ys_h, segs_o, cnts_o)
    run_rel(sp_h, dp_h, yp_h, segp_o, cntp_o)


def _sc_scatter(yd, ys, yp, sd, dd, ss, ds, sp, dp):
    mesh = plsc.VectorSubcoreMesh(core_axis_name="c", subcore_axis_name="s")
    f32 = jnp.float32
    ones = jnp.ones((CHUNK, CW), f32)
    zseg = jnp.zeros((SEG_PER_TILE, H), f32)
    zcnt = jnp.zeros((SEG_PER_TILE, CW), f32)
    call = pl.kernel(
        _sc_body,
        out_type=[jax.ShapeDtypeStruct((NC, ACC_ROWS, H), f32)] * 3
                 + [jax.ShapeDtypeStruct((NC, ACC_ROWS, CW), f32)] * 3,
        mesh=mesh,
        compiler_params=pltpu.CompilerParams(use_tc_tiling_on_sc=False),
        scratch_types=[
            pltpu.VMEM((ROWS_PER_TILE, CHUNK), jnp.int32),
            pltpu.VMEM((ROWS_PER_TILE, CHUNK), jnp.int32),
            pltpu.VMEM((NBUF, CHUNK, H), f32),
            pltpu.VMEM((CHUNK, CW), f32),
            pltpu.VMEM_SHARED((ACC_ROWS, H), f32),
            pltpu.VMEM_SHARED((ACC_ROWS, CW), f32),
        ] + [pltpu.SemaphoreType.DMA] * (2 * NBUF),
    )
    return call(yd, ys, yp, sd, dd, ss, ds, sp, dp, ones, zseg, zcnt)


def _comb_body(segd, segs, segp, cntd, cnts, cntp, ru, rp, uo, po):
    def mean(seg_ref, cnt_ref):
        s = seg_ref[0] + seg_ref[1]
        c = cnt_ref[0][:, 0:1] + cnt_ref[1][:, 0:1]
        return s / jnp.maximum(c, 1.0)

    uo[...] = jnp.maximum(
        mean(segd, cntd) + 0.75 * mean(segs, cnts) + ru[...], 0.0)
    po[...] = jnp.maximum(mean(segp, cntp) + rp[...], 0.0)


def _combine(segd, segs, segp, cntd, cnts, cntp, ru, rp):
    n_blk = N_NODE // BLK
    seg_spec = pl.BlockSpec((NC, BLK, H), lambda i: (0, i, 0))
    cnt_spec = pl.BlockSpec((NC, BLK, CW), lambda i: (0, i, 0))
    r_spec = pl.BlockSpec((BLK, H), lambda i: (i, 0))
    out_shape = jax.ShapeDtypeStruct((N_NODE, H), jnp.float32)
    return pl.pallas_call(
        _comb_body,
        grid=(n_blk,),
        in_specs=[seg_spec] * 3 + [cnt_spec] * 3 + [r_spec] * 2,
        out_specs=[r_spec] * 2,
        out_shape=[out_shape] * 2,
    )(segd, segs, segp, cntd, cnts, cntp, ru, rp)


def kernel(user_x, post_x, edge_index_rev_engages, edge_index_social,
           edge_index_engages, Wl_d, bl_d, Wr_d, Wl_s, bl_s, Wr_s,
           Wl_p, bl_p, Wr_p):
    yd, ys, yp, ru, rp = _project(user_x, post_x, Wl_d, Wl_s, Wl_p,
                                  Wr_d, Wr_s, Wr_p, bl_d, bl_s, bl_p)

    pad = E_PAD - E

    def prep(ei):
        src = jnp.concatenate(
            [ei[0], jnp.zeros((pad,), jnp.int32)]).reshape(N_ROWS, CHUNK)
        dst = jnp.concatenate(
            [ei[1], jnp.full((pad,), DUMMY, jnp.int32)]).reshape(N_ROWS, CHUNK)
        return src, dst

    sd, dd = prep(edge_index_rev_engages)
    ss, ds = prep(edge_index_social)
    sp, dp = prep(edge_index_engages)

    segd, segs, segp, cntd, cnts, cntp = _sc_scatter(
        yd, ys, yp, sd, dd, ss, ds, sp, dp)

    user_out, post_out = _combine(segd, segs, segp, cntd, cnts, cntp, ru, rp)
    return (user_out, post_out)


# speedup vs baseline: 1.0170x; 1.0170x over previous
import functools

import jax
import jax.numpy as jnp
from jax import lax
from jax.experimental import pallas as pl
from jax.experimental.pallas import tpu as pltpu
from jax.experimental.pallas import tpu_sc as plsc

N_NODE = 10000
D_IN = 128
H = 64
E = 320000

NC = 2
NS = 16
TILES = NC * NS
CHUNK = 128
ROWS_PER_TILE = 80
N_ROWS = TILES * ROWS_PER_TILE
E_PAD = N_ROWS * CHUNK
DUMMY = N_NODE
ACC_ROWS = 10240
SEG_PER_TILE = ACC_ROWS // NS
CW = 16
NBUF = 5
AHEAD = 3

BLK = 200


def _proj_body(u_ref, p_ref, wld, wls, wlp, wrd, wrs, wrp, bld, bls, blp,
               yd, ys, yp, ru, rp):
    u = u_ref[...]
    p = p_ref[...]
    f32 = jnp.float32
    yd[...] = jnp.dot(p, wld[...], preferred_element_type=f32)
    ys[...] = jnp.dot(u, wls[...], preferred_element_type=f32)
    yp[...] = jnp.dot(u, wlp[...], preferred_element_type=f32)
    ru[...] = (jnp.dot(u, wrd[...], preferred_element_type=f32)
               + 0.75 * jnp.dot(u, wrs[...], preferred_element_type=f32)
               + bld[...] + 0.75 * bls[...])
    rp[...] = jnp.dot(p, wrp[...], preferred_element_type=f32) + blp[...]


def _project(user_x, post_x, Wl_d, Wl_s, Wl_p, Wr_d, Wr_s, Wr_p, bl_d, bl_s, bl_p):
    n_blk = N_NODE // BLK
    row_spec = pl.BlockSpec((BLK, D_IN), lambda i: (i, 0))
    w_spec = pl.BlockSpec((D_IN, H), lambda i: (0, 0))
    b_spec = pl.BlockSpec((1, H), lambda i: (0, 0))
    out_spec = pl.BlockSpec((BLK, H), lambda i: (i, 0))
    out_shape = jax.ShapeDtypeStruct((N_NODE, H), jnp.float32)
    return pl.pallas_call(
        _proj_body,
        grid=(n_blk,),
        in_specs=[row_spec, row_spec] + [w_spec] * 6 + [b_spec] * 3,
        out_specs=[out_spec] * 5,
        out_shape=[out_shape] * 5,
    )(user_x, post_x, Wl_d, Wl_s, Wl_p, Wr_d, Wr_s, Wr_p,
      bl_d.reshape(1, H), bl_s.reshape(1, H), bl_p.reshape(1, H))


def _sc_body(yd_h, ys_h, yp_h, sd_h, dd_h, ss_h, ds_h, sp_h, dp_h,
             ones_h, zseg_h, zcnt_h,
             segd_o, segs_o, segp_o, cntd_o, cnts_o, cntp_o,
             src_v, dst_v, rows_v, ones_v,
             seg_sh, cnt_sh,
             g0, g1, g2, g3, g4, s0, s1, s2, s3, s4):
    cid = lax.axis_index("c")
    sid = lax.axis_index("s")
    tid = sid * NC + cid
    g_sems = (g0, g1, g2, g3, g4)
    s_sems = (s0, s1, s2, s3, s4)

    pltpu.sync_copy(ones_h, ones_v)

    my_acc = pl.ds(sid * SEG_PER_TILE, SEG_PER_TILE)

    def run_rel(src_h, dst_h, y_h, seg_o, cnt_o):
        pltpu.sync_copy(zseg_h, seg_sh.at[my_acc])
        pltpu.sync_copy(zcnt_h, cnt_sh.at[my_acc])
        row0 = tid * ROWS_PER_TILE
        pltpu.sync_copy(src_h.at[pl.ds(row0, ROWS_PER_TILE)], src_v)
        pltpu.sync_copy(dst_h.at[pl.ds(row0, ROWS_PER_TILE)], dst_v)
        plsc.subcore_barrier()

        def start_gather(row, b):
            pltpu.async_copy(y_h.at[src_v.at[row]], rows_v.at[b], g_sems[b])

        def wait_gather(row, b):
            pltpu.make_async_copy(y_h.at[src_v.at[row]], rows_v.at[b],
                                  g_sems[b]).wait()

        def start_scatter(row, b):
            pltpu.async_copy(rows_v.at[b], seg_sh.at[dst_v.at[row]],
                             s_sems[b], add=True)
            pltpu.async_copy(ones_v, cnt_sh.at[dst_v.at[row]],
                             s_sems[b], add=True)

        def wait_scatter(row, b):
            pltpu.make_async_copy(rows_v.at[b], seg_sh.at[dst_v.at[row]],
                                  s_sems[b]).wait()
            pltpu.make_async_copy(ones_v, cnt_sh.at[dst_v.at[row]],
                                  s_sems[b]).wait()

        for b in range(AHEAD):
            start_gather(b, b)

        def step(jn, _):
            j0 = jn * NBUF
            for b in range(NBUF):
                j = j0 + b
                wait_gather(j, b)
                start_scatter(j, b)
                bn = (b + AHEAD) % NBUF

                @pl.when(j >= NBUF - AHEAD)
                def _():
                    wait_scatter(j - (NBUF - AHEAD), bn)

                @pl.when(j + AHEAD < ROWS_PER_TILE)
                def _():
                    start_gather(j + AHEAD, bn)
            return _

        lax.fori_loop(0, ROWS_PER_TILE // NBUF, step, None)
        for j in range(ROWS_PER_TILE - (NBUF - AHEAD), ROWS_PER_TILE):
            wait_scatter(j, j % NBUF)
        plsc.subcore_barrier()
        pltpu.sync_copy(seg_sh.at[my_acc], seg_o.at[cid, my_acc])
        pltpu.sync_copy(cnt_sh.at[my_acc], cnt_o.at[cid, my_acc])

    run_rel(sd_h, dd_h, yd_h, segd_o, cntd_o)
    run_rel(ss_h, ds_h, ys_h, segs_o, cnts_o)
    run_rel(sp_h, dp_h, yp_h, segp_o, cntp_o)


def _sc_scatter(yd, ys, yp, sd, dd, ss, ds, sp, dp):
    mesh = plsc.VectorSubcoreMesh(core_axis_name="c", subcore_axis_name="s")
    f32 = jnp.float32
    ones = jnp.ones((CHUNK, CW), f32)
    zseg = jnp.zeros((SEG_PER_TILE, H), f32)
    zcnt = jnp.zeros((SEG_PER_TILE, CW), f32)
    call = pl.kernel(
        _sc_body,
        out_type=[jax.ShapeDtypeStruct((NC, ACC_ROWS, H), f32)] * 3
                 + [jax.ShapeDtypeStruct((NC, ACC_ROWS, CW), f32)] * 3,
        mesh=mesh,
        compiler_params=pltpu.CompilerParams(use_tc_tiling_on_sc=False),
        scratch_types=[
            pltpu.VMEM((ROWS_PER_TILE, CHUNK), jnp.int32),
            pltpu.VMEM((ROWS_PER_TILE, CHUNK), jnp.int32),
            pltpu.VMEM((NBUF, CHUNK, H), f32),
            pltpu.VMEM((CHUNK, CW), f32),
            pltpu.VMEM_SHARED((ACC_ROWS, H), f32),
            pltpu.VMEM_SHARED((ACC_ROWS, CW), f32),
        ] + [pltpu.SemaphoreType.DMA] * (2 * NBUF),
    )
    return call(yd, ys, yp, sd, dd, ss, ds, sp, dp, ones, zseg, zcnt)


def _comb_body(segd, segs, segp, cntd, cnts, cntp, ru, rp, uo, po):
    def mean(seg_ref, cnt_ref):
        s = seg_ref[0] + seg_ref[1]
        c = cnt_ref[0][:, 0:1] + cnt_ref[1][:, 0:1]
        return s / jnp.maximum(c, 1.0)

    uo[...] = jnp.maximum(
        mean(segd, cntd) + 0.75 * mean(segs, cnts) + ru[...], 0.0)
    po[...] = jnp.maximum(mean(segp, cntp) + rp[...], 0.0)


def _combine(segd, segs, segp, cntd, cnts, cntp, ru, rp):
    n_blk = N_NODE // BLK
    seg_spec = pl.BlockSpec((NC, BLK, H), lambda i: (0, i, 0))
    cnt_spec = pl.BlockSpec((NC, BLK, CW), lambda i: (0, i, 0))
    r_spec = pl.BlockSpec((BLK, H), lambda i: (i, 0))
    out_shape = jax.ShapeDtypeStruct((N_NODE, H), jnp.float32)
    return pl.pallas_call(
        _comb_body,
        grid=(n_blk,),
        in_specs=[seg_spec] * 3 + [cnt_spec] * 3 + [r_spec] * 2,
        out_specs=[r_spec] * 2,
        out_shape=[out_shape] * 2,
    )(segd, segs, segp, cntd, cnts, cntp, ru, rp)


def kernel(user_x, post_x, edge_index_rev_engages, edge_index_social,
           edge_index_engages, Wl_d, bl_d, Wr_d, Wl_s, bl_s, Wr_s,
           Wl_p, bl_p, Wr_p):
    yd, ys, yp, ru, rp = _project(user_x, post_x, Wl_d, Wl_s, Wl_p,
                                  Wr_d, Wr_s, Wr_p, bl_d, bl_s, bl_p)

    pad = E_PAD - E

    def prep(ei):
        src = jnp.concatenate(
            [ei[0], jnp.zeros((pad,), jnp.int32)]).reshape(N_ROWS, CHUNK)
        dst = jnp.concatenate(
            [ei[1], jnp.full((pad,), DUMMY, jnp.int32)]).reshape(N_ROWS, CHUNK)
        return src, dst

    sd, dd = prep(edge_index_rev_engages)
    ss, ds = prep(edge_index_social)
    sp, dp = prep(edge_index_engages)

    segd, segs, segp, cntd, cnts, cntp = _sc_scatter(
        yd, ys, yp, sd, dd, ss, ds, sp, dp)

    user_out, post_out = _combine(segd, segs, segp, cntd, cnts, cntp, ru, rp)
    return (user_out, post_out)


# speedup vs baseline: 2.3523x; 2.3131x over previous
import functools

import jax
import jax.numpy as jnp
from jax import lax
from jax.experimental import pallas as pl
from jax.experimental.pallas import tpu as pltpu
from jax.experimental.pallas import tpu_sc as plsc

N_NODE = 10000
D_IN = 128
H = 64
E = 320000

NC = 2
NS = 16
TILES = NC * NS
CHUNK = 128
ROWS_PER_TILE = 80
N_ROWS = TILES * ROWS_PER_TILE
E_PAD = N_ROWS * CHUNK
DUMMY = N_NODE
ACC_ROWS = 10240
SEG_PER_TILE = ACC_ROWS // NS
CW = 16
NBUF = 5
AHEAD = 3

BLK = 200


def _proj_body(u_ref, p_ref, wld, wls, wlp, wrd, wrs, wrp, bld, bls, blp,
               yd, ys, yp, ru, rp):
    u = u_ref[...]
    p = p_ref[...]
    f32 = jnp.float32
    yd[...] = jnp.dot(p, wld[...], preferred_element_type=f32)
    ys[...] = jnp.dot(u, wls[...], preferred_element_type=f32)
    yp[...] = jnp.dot(u, wlp[...], preferred_element_type=f32)
    ru[...] = (jnp.dot(u, wrd[...], preferred_element_type=f32)
               + 0.75 * jnp.dot(u, wrs[...], preferred_element_type=f32)
               + bld[...] + 0.75 * bls[...])
    rp[...] = jnp.dot(p, wrp[...], preferred_element_type=f32) + blp[...]


def _project(user_x, post_x, Wl_d, Wl_s, Wl_p, Wr_d, Wr_s, Wr_p, bl_d, bl_s, bl_p):
    n_blk = N_NODE // BLK
    row_spec = pl.BlockSpec((BLK, D_IN), lambda i: (i, 0))
    w_spec = pl.BlockSpec((D_IN, H), lambda i: (0, 0))
    b_spec = pl.BlockSpec((1, H), lambda i: (0, 0))
    out_spec = pl.BlockSpec((BLK, H), lambda i: (i, 0))
    out_shape = jax.ShapeDtypeStruct((N_NODE, H), jnp.float32)
    return pl.pallas_call(
        _proj_body,
        grid=(n_blk,),
        in_specs=[row_spec, row_spec] + [w_spec] * 6 + [b_spec] * 3,
        out_specs=[out_spec] * 5,
        out_shape=[out_shape] * 5,
    )(user_x, post_x, Wl_d, Wl_s, Wl_p, Wr_d, Wr_s, Wr_p,
      bl_d.reshape(1, H), bl_s.reshape(1, H), bl_p.reshape(1, H))


def _sc_body(yd_h, ys_h, yp_h, sd_h, dd_h, ss_h, ds_h, sp_h, dp_h,
             ones_h, zseg_h, zcnt_h,
             segd_o, segs_o, segp_o, cntd_o, cnts_o, cntp_o,
             src_v, dst_v, rows_v, ones_v,
             seg_sh, cnt_sh,
             g0, g1, g2, g3, g4, s0, s1, s2, s3, s4):
    cid = lax.axis_index("c")
    sid = lax.axis_index("s")
    tid = sid * NC + cid
    g_sems = (g0, g1, g2, g3, g4)
    s_sems = (s0, s1, s2, s3, s4)

    pltpu.sync_copy(ones_h, ones_v)

    my_acc = pl.ds(sid * SEG_PER_TILE, SEG_PER_TILE)

    def run_rel(src_h, dst_h, y_h, seg_o, cnt_o):
        pltpu.sync_copy(zseg_h, seg_sh.at[my_acc])
        pltpu.sync_copy(zcnt_h, cnt_sh.at[my_acc])
        row0 = tid * ROWS_PER_TILE
        pltpu.sync_copy(src_h.at[pl.ds(row0, ROWS_PER_TILE)], src_v)
        pltpu.sync_copy(dst_h.at[pl.ds(row0, ROWS_PER_TILE)], dst_v)
        plsc.subcore_barrier()

        def start_gather(row, b):
            pltpu.async_copy(y_h.at[src_v.at[row]], rows_v.at[b], g_sems[b])

        def wait_gather(row, b):
            pltpu.make_async_copy(y_h.at[src_v.at[row]], rows_v.at[b],
                                  g_sems[b]).wait()

        def start_scatter(row, b):
            pltpu.async_copy(rows_v.at[b], seg_sh.at[dst_v.at[row]],
                             s_sems[b], add=True)
            pltpu.async_copy(ones_v, cnt_sh.at[dst_v.at[row]],
                             s_sems[b], add=True)

        def wait_scatter(row, b):
            pltpu.make_async_copy(rows_v.at[b], seg_sh.at[dst_v.at[row]],
                                  s_sems[b]).wait()
            pltpu.make_async_copy(ones_v, cnt_sh.at[dst_v.at[row]],
                                  s_sems[b]).wait()

        for b in range(AHEAD):
            start_gather(b, b)

        def step(jn, _):
            j0 = jn * NBUF
            for b in range(NBUF):
                j = j0 + b
                wait_gather(j, b)
                start_scatter(j, b)
                bn = (b + AHEAD) % NBUF

                @pl.when(j >= NBUF - AHEAD)
                def _():
                    wait_scatter(j - (NBUF - AHEAD), bn)

                @pl.when(j + AHEAD < ROWS_PER_TILE)
                def _():
                    start_gather(j + AHEAD, bn)
            return _

        lax.fori_loop(0, ROWS_PER_TILE // NBUF, step, None)
        for j in range(ROWS_PER_TILE - (NBUF - AHEAD), ROWS_PER_TILE):
            wait_scatter(j, j % NBUF)
        plsc.subcore_barrier()
        pltpu.sync_copy(seg_sh.at[my_acc], seg_o.at[cid, my_acc])
        pltpu.sync_copy(cnt_sh.at[my_acc], cnt_o.at[cid, my_acc])

    run_rel(sd_h, dd_h, yd_h, segd_o, cntd_o)
    run_rel(ss_h, ds_h, ys_h, segs_o, cnts_o)
    run_rel(sp_h, dp_h, yp_h, segp_o, cntp_o)


def _sc_scatter(yd, ys, yp, sd, dd, ss, ds, sp, dp):
    mesh = plsc.VectorSubcoreMesh(core_axis_name="c", subcore_axis_name="s")
    f32 = jnp.float32
    ones = jnp.ones((CHUNK, CW), f32)
    zseg = jnp.zeros((SEG_PER_TILE, H), f32)
    zcnt = jnp.zeros((SEG_PER_TILE, CW), f32)
    call = pl.kernel(
        _sc_body,
        out_type=[jax.ShapeDtypeStruct((NC, ACC_ROWS, H), f32)] * 3
                 + [jax.ShapeDtypeStruct((NC, ACC_ROWS, CW), f32)] * 3,
        mesh=mesh,
        compiler_params=pltpu.CompilerParams(use_tc_tiling_on_sc=False),
        scratch_types=[
            pltpu.VMEM((ROWS_PER_TILE, CHUNK), jnp.int32),
            pltpu.VMEM((ROWS_PER_TILE, CHUNK), jnp.int32),
            pltpu.VMEM((NBUF, CHUNK, H), f32),
            pltpu.VMEM((CHUNK, CW), f32),
            pltpu.VMEM_SHARED((ACC_ROWS, H), f32),
            pltpu.VMEM_SHARED((ACC_ROWS, CW), f32),
        ] + [pltpu.SemaphoreType.DMA] * (2 * NBUF),
    )
    return call(yd, ys, yp, sd, dd, ss, ds, sp, dp, ones, zseg, zcnt)


def _comb_body(segd, segs, segp, cntd, cnts, cntp, ru, rp, uo, po):
    def mean(seg_ref, cnt_ref):
        s = seg_ref[0] + seg_ref[1]
        c = cnt_ref[0][:, 0:1] + cnt_ref[1][:, 0:1]
        return s / jnp.maximum(c, 1.0)

    uo[...] = jnp.maximum(
        mean(segd, cntd) + 0.75 * mean(segs, cnts) + ru[...], 0.0)
    po[...] = jnp.maximum(mean(segp, cntp) + rp[...], 0.0)


def _combine(segd, segs, segp, cntd, cnts, cntp, ru, rp):
    n_blk = N_NODE // BLK
    seg_spec = pl.BlockSpec((NC, BLK, H), lambda i: (0, i, 0))
    cnt_spec = pl.BlockSpec((NC, BLK, CW), lambda i: (0, i, 0))
    r_spec = pl.BlockSpec((BLK, H), lambda i: (i, 0))
    out_shape = jax.ShapeDtypeStruct((N_NODE, H), jnp.float32)
    return pl.pallas_call(
        _comb_body,
        grid=(n_blk,),
        in_specs=[seg_spec] * 3 + [cnt_spec] * 3 + [r_spec] * 2,
        out_specs=[r_spec] * 2,
        out_shape=[out_shape] * 2,
    )(segd, segs, segp, cntd, cnts, cntp, ru, rp)


def kernel(user_x, post_x, edge_index_rev_engages, edge_index_social,
           edge_index_engages, Wl_d, bl_d, Wr_d, Wl_s, bl_s, Wr_s,
           Wl_p, bl_p, Wr_p):
    yd, ys, yp, ru, rp = _project(user_x, post_x, Wl_d, Wl_s, Wl_p,
                                  Wr_d, Wr_s, Wr_p, bl_d, bl_s, bl_p)

    pad = E_PAD - E
    pad_src = (jnp.arange(pad, dtype=jnp.int32) * 37) % N_NODE
    pad_dst = DUMMY + (jnp.arange(pad, dtype=jnp.int32) % (ACC_ROWS - DUMMY))

    def prep(ei):
        src = jnp.concatenate([ei[0], pad_src]).reshape(N_ROWS, CHUNK)
        dst = jnp.concatenate([ei[1], pad_dst]).reshape(N_ROWS, CHUNK)
        return src, dst

    sd, dd = prep(edge_index_rev_engages)
    ss, ds = prep(edge_index_social)
    sp, dp = prep(edge_index_engages)

    segd, segs, segp, cntd, cnts, cntp = _sc_scatter(
        yd, ys, yp, sd, dd, ss, ds, sp, dp)

    user_out, post_out = _combine(segd, segs, segp, cntd, cnts, cntp, ru, rp)
    return (user_out, post_out)


# speedup vs baseline: 2.5948x; 1.1031x over previous
import functools

import jax
import jax.numpy as jnp
from jax import lax
from jax.experimental import pallas as pl
from jax.experimental.pallas import tpu as pltpu
from jax.experimental.pallas import tpu_sc as plsc

N_NODE = 10000
D_IN = 128
H = 64
E = 320000

NC = 2
NS = 16
TILES = NC * NS
CHUNK = 128
N_ROWS = E // CHUNK
BASE_ROWS = N_ROWS // TILES
EXTRA = N_ROWS - BASE_ROWS * TILES
ROWS_MAX = BASE_ROWS + 1
ROWS_LOOP = 80
ACC_ROWS = 10240
SEG_PER_TILE = ACC_ROWS // NS
CW = 16
NBUF = 5
AHEAD = 3

BLK = 200


def _proj_body(u_ref, p_ref, wld, wls, wlp, wrd, wrs, wrp, bld, bls, blp,
               yd, ys, yp, ru, rp):
    u = u_ref[...]
    p = p_ref[...]
    f32 = jnp.float32
    yd[...] = jnp.dot(p, wld[...], preferred_element_type=f32)
    ys[...] = jnp.dot(u, wls[...], preferred_element_type=f32)
    yp[...] = jnp.dot(u, wlp[...], preferred_element_type=f32)
    ru[...] = (jnp.dot(u, wrd[...], preferred_element_type=f32)
               + 0.75 * jnp.dot(u, wrs[...], preferred_element_type=f32)
               + bld[...] + 0.75 * bls[...])
    rp[...] = jnp.dot(p, wrp[...], preferred_element_type=f32) + blp[...]


def _project(user_x, post_x, Wl_d, Wl_s, Wl_p, Wr_d, Wr_s, Wr_p, bl_d, bl_s, bl_p):
    n_blk = N_NODE // BLK
    row_spec = pl.BlockSpec((BLK, D_IN), lambda i: (i, 0))
    w_spec = pl.BlockSpec((D_IN, H), lambda i: (0, 0))
    b_spec = pl.BlockSpec((1, H), lambda i: (0, 0))
    out_spec = pl.BlockSpec((BLK, H), lambda i: (i, 0))
    out_shape = jax.ShapeDtypeStruct((N_NODE, H), jnp.float32)
    return pl.pallas_call(
        _proj_body,
        grid=(n_blk,),
        in_specs=[row_spec, row_spec] + [w_spec] * 6 + [b_spec] * 3,
        out_specs=[out_spec] * 5,
        out_shape=[out_shape] * 5,
    )(user_x, post_x, Wl_d, Wl_s, Wl_p, Wr_d, Wr_s, Wr_p,
      bl_d.reshape(1, H), bl_s.reshape(1, H), bl_p.reshape(1, H))


def _sc_body(yd_h, ys_h, yp_h, ed_h, es_h, ep_h,
             ones_h, zseg_h, zcnt_h,
             segd_o, segs_o, segp_o, cntd_o, cnts_o, cntp_o,
             src_v, dst_v, rows_v, ones_v,
             seg_sh, cnt_sh,
             g0, g1, g2, g3, g4, s0, s1, s2, s3, s4):
    cid = lax.axis_index("c")
    sid = lax.axis_index("s")
    tid = sid * NC + cid
    g_sems = (g0, g1, g2, g3, g4)
    s_sems = (s0, s1, s2, s3, s4)

    row0 = tid * BASE_ROWS + jnp.minimum(tid, EXTRA)
    nrows = jnp.where(tid < EXTRA, BASE_ROWS + 1, BASE_ROWS)

    pltpu.sync_copy(ones_h, ones_v)

    my_acc = pl.ds(sid * SEG_PER_TILE, SEG_PER_TILE)

    def run_rel(e_h, y_h, seg_o, cnt_o):
        pltpu.sync_copy(zseg_h, seg_sh.at[my_acc])
        pltpu.sync_copy(zcnt_h, cnt_sh.at[my_acc])
        pltpu.sync_copy(e_h.at[0, pl.ds(row0, BASE_ROWS)],
                        src_v.at[pl.ds(0, BASE_ROWS)])
        pltpu.sync_copy(e_h.at[1, pl.ds(row0, BASE_ROWS)],
                        dst_v.at[pl.ds(0, BASE_ROWS)])

        @pl.when(tid < EXTRA)
        def _():
            pltpu.sync_copy(e_h.at[0, row0 + BASE_ROWS], src_v.at[BASE_ROWS])
            pltpu.sync_copy(e_h.at[1, row0 + BASE_ROWS], dst_v.at[BASE_ROWS])

        plsc.subcore_barrier()

        def start_gather(row, b):
            pltpu.async_copy(y_h.at[src_v.at[row]], rows_v.at[b], g_sems[b])

        def wait_gather(row, b):
            pltpu.make_async_copy(y_h.at[src_v.at[row]], rows_v.at[b],
                                  g_sems[b]).wait()

        def start_scatter(row, b):
            pltpu.async_copy(rows_v.at[b], seg_sh.at[dst_v.at[row]],
                             s_sems[b], add=True)
            pltpu.async_copy(ones_v, cnt_sh.at[dst_v.at[row]],
                             s_sems[b], add=True)

        def wait_scatter(row, b):
            pltpu.make_async_copy(rows_v.at[b], seg_sh.at[dst_v.at[row]],
                                  s_sems[b]).wait()
            pltpu.make_async_copy(ones_v, cnt_sh.at[dst_v.at[row]],
                                  s_sems[b]).wait()

        for b in range(AHEAD):
            start_gather(b, b)

        def step(jn, _):
            j0 = jn * NBUF
            for b in range(NBUF):
                j = j0 + b
                live = j < nrows

                @pl.when(live)
                def _():
                    wait_gather(j, b)
                    start_scatter(j, b)

                bn = (b + AHEAD) % NBUF

                @pl.when((j >= NBUF - AHEAD)
                         & (j - (NBUF - AHEAD) < nrows))
                def _():
                    wait_scatter(j - (NBUF - AHEAD), bn)

                @pl.when(j + AHEAD < nrows)
                def _():
                    start_gather(j + AHEAD, bn)
            return _

        lax.fori_loop(0, ROWS_LOOP // NBUF, step, None)
        for j in range(ROWS_LOOP - (NBUF - AHEAD), ROWS_LOOP):
            @pl.when(j < nrows)
            def _():
                wait_scatter(j, j % NBUF)
        plsc.subcore_barrier()
        pltpu.sync_copy(seg_sh.at[my_acc], seg_o.at[cid, my_acc])
        pltpu.sync_copy(cnt_sh.at[my_acc], cnt_o.at[cid, my_acc])

    run_rel(ed_h, yd_h, segd_o, cntd_o)
    run_rel(es_h, ys_h, segs_o, cnts_o)
    run_rel(ep_h, yp_h, segp_o, cntp_o)


def _sc_scatter(yd, ys, yp, ed, es, ep):
    mesh = plsc.VectorSubcoreMesh(core_axis_name="c", subcore_axis_name="s")
    f32 = jnp.float32
    ones = jnp.ones((CHUNK, CW), f32)
    zseg = jnp.zeros((SEG_PER_TILE, H), f32)
    zcnt = jnp.zeros((SEG_PER_TILE, CW), f32)
    call = pl.kernel(
        _sc_body,
        out_type=[jax.ShapeDtypeStruct((NC, ACC_ROWS, H), f32)] * 3
                 + [jax.ShapeDtypeStruct((NC, ACC_ROWS, CW), f32)] * 3,
        mesh=mesh,
        compiler_params=pltpu.CompilerParams(use_tc_tiling_on_sc=False),
        scratch_types=[
            pltpu.VMEM((ROWS_MAX, CHUNK), jnp.int32),
            pltpu.VMEM((ROWS_MAX, CHUNK), jnp.int32),
            pltpu.VMEM((NBUF, CHUNK, H), f32),
            pltpu.VMEM((CHUNK, CW), f32),
            pltpu.VMEM_SHARED((ACC_ROWS, H), f32),
            pltpu.VMEM_SHARED((ACC_ROWS, CW), f32),
        ] + [pltpu.SemaphoreType.DMA] * (2 * NBUF),
    )
    return call(yd, ys, yp, ed, es, ep, ones, zseg, zcnt)


def _comb_body(segd, segs, segp, cntd, cnts, cntp, ru, rp, uo, po):
    def mean(seg_ref, cnt_ref):
        s = seg_ref[0] + seg_ref[1]
        c = cnt_ref[0][:, 0:1] + cnt_ref[1][:, 0:1]
        return s / jnp.maximum(c, 1.0)

    uo[...] = jnp.maximum(
        mean(segd, cntd) + 0.75 * mean(segs, cnts) + ru[...], 0.0)
    po[...] = jnp.maximum(mean(segp, cntp) + rp[...], 0.0)


def _combine(segd, segs, segp, cntd, cnts, cntp, ru, rp):
    n_blk = N_NODE // BLK
    seg_spec = pl.BlockSpec((NC, BLK, H), lambda i: (0, i, 0))
    cnt_spec = pl.BlockSpec((NC, BLK, CW), lambda i: (0, i, 0))
    r_spec = pl.BlockSpec((BLK, H), lambda i: (i, 0))
    out_shape = jax.ShapeDtypeStruct((N_NODE, H), jnp.float32)
    return pl.pallas_call(
        _comb_body,
        grid=(n_blk,),
        in_specs=[seg_spec] * 3 + [cnt_spec] * 3 + [r_spec] * 2,
        out_specs=[r_spec] * 2,
        out_shape=[out_shape] * 2,
    )(segd, segs, segp, cntd, cnts, cntp, ru, rp)


def kernel(user_x, post_x, edge_index_rev_engages, edge_index_social,
           edge_index_engages, Wl_d, bl_d, Wr_d, Wl_s, bl_s, Wr_s,
           Wl_p, bl_p, Wr_p):
    yd, ys, yp, ru, rp = _project(user_x, post_x, Wl_d, Wl_s, Wl_p,
                                  Wr_d, Wr_s, Wr_p, bl_d, bl_s, bl_p)

    ed = edge_index_rev_engages.reshape(2, N_ROWS, CHUNK)
    es = edge_index_social.reshape(2, N_ROWS, CHUNK)
    ep = edge_index_engages.reshape(2, N_ROWS, CHUNK)

    segd, segs, segp, cntd, cnts, cntp = _sc_scatter(yd, ys, yp, ed, es, ep)

    user_out, post_out = _combine(segd, segs, segp, cntd, cnts, cntp, ru, rp)
    return (user_out, post_out)


# speedup vs baseline: 2.7927x; 1.0763x over previous
import functools

import jax
import jax.numpy as jnp
from jax import lax
from jax.experimental import pallas as pl
from jax.experimental.pallas import tpu as pltpu
from jax.experimental.pallas import tpu_sc as plsc

N_NODE = 10000
D_IN = 128
H = 64
E = 320000

NC = 2
NS = 16
TILES = NC * NS
CHUNK = 128
N_ROWS = E // CHUNK
BASE_ROWS = N_ROWS // TILES
EXTRA = N_ROWS - BASE_ROWS * TILES
ROWS_MAX = BASE_ROWS + 1
ROWS_LOOP = 80
ACC_ROWS = 10240
SEG_PER_TILE = ACC_ROWS // NS
CW = 16
NBUF = 5
AHEAD = 3

BLK = 400


def _proj_body(u_ref, p_ref, wld, wls, wlp, wrd, wrs, wrp, bld, bls, blp,
               yd, ys, yp, ru, rp):
    u = u_ref[...]
    p = p_ref[...]
    f32 = jnp.float32
    yd[...] = jnp.dot(p, wld[...], preferred_element_type=f32)
    ys[...] = jnp.dot(u, wls[...], preferred_element_type=f32)
    yp[...] = jnp.dot(u, wlp[...], preferred_element_type=f32)
    ru[...] = (jnp.dot(u, wrd[...], preferred_element_type=f32)
               + 0.75 * jnp.dot(u, wrs[...], preferred_element_type=f32)
               + bld[...] + 0.75 * bls[...])
    rp[...] = jnp.dot(p, wrp[...], preferred_element_type=f32) + blp[...]


def _project(user_x, post_x, Wl_d, Wl_s, Wl_p, Wr_d, Wr_s, Wr_p, bl_d, bl_s, bl_p):
    n_blk = N_NODE // BLK
    row_spec = pl.BlockSpec((BLK, D_IN), lambda i: (i, 0))
    w_spec = pl.BlockSpec((D_IN, H), lambda i: (0, 0))
    b_spec = pl.BlockSpec((1, H), lambda i: (0, 0))
    out_spec = pl.BlockSpec((BLK, H), lambda i: (i, 0))
    out_shape = jax.ShapeDtypeStruct((N_NODE, H), jnp.float32)
    return pl.pallas_call(
        _proj_body,
        grid=(n_blk,),
        in_specs=[row_spec, row_spec] + [w_spec] * 6 + [b_spec] * 3,
        out_specs=[out_spec] * 5,
        out_shape=[out_shape] * 5,
    )(user_x, post_x, Wl_d, Wl_s, Wl_p, Wr_d, Wr_s, Wr_p,
      bl_d.reshape(1, H), bl_s.reshape(1, H), bl_p.reshape(1, H))


def _sc_body(yd_h, ys_h, yp_h, ed_h, es_h, ep_h,
             ones_h, zseg_h, zcnt_h,
             segd_o, segs_o, segp_o, cntd_o, cnts_o, cntp_o,
             src_v, dst_v, rows_v, ones_v,
             seg_sh, cnt_sh,
             g0, g1, g2, g3, g4, s0, s1, s2, s3, s4):
    cid = lax.axis_index("c")
    sid = lax.axis_index("s")
    tid = sid * NC + cid
    g_sems = (g0, g1, g2, g3, g4)
    s_sems = (s0, s1, s2, s3, s4)

    row0 = tid * BASE_ROWS + jnp.minimum(tid, EXTRA)
    nrows = jnp.where(tid < EXTRA, BASE_ROWS + 1, BASE_ROWS)

    pltpu.sync_copy(ones_h, ones_v)

    my_acc = pl.ds(sid * SEG_PER_TILE, SEG_PER_TILE)

    def run_rel(e_h, y_h, seg_o, cnt_o):
        pltpu.sync_copy(zseg_h, seg_sh.at[my_acc])
        pltpu.sync_copy(zcnt_h, cnt_sh.at[my_acc])
        pltpu.sync_copy(e_h.at[0, pl.ds(row0, BASE_ROWS)],
                        src_v.at[pl.ds(0, BASE_ROWS)])
        pltpu.sync_copy(e_h.at[1, pl.ds(row0, BASE_ROWS)],
                        dst_v.at[pl.ds(0, BASE_ROWS)])

        @pl.when(tid < EXTRA)
        def _():
            pltpu.sync_copy(e_h.at[0, row0 + BASE_ROWS], src_v.at[BASE_ROWS])
            pltpu.sync_copy(e_h.at[1, row0 + BASE_ROWS], dst_v.at[BASE_ROWS])

        plsc.subcore_barrier()

        def start_gather(row, b):
            pltpu.async_copy(y_h.at[src_v.at[row]], rows_v.at[b], g_sems[b])

        def wait_gather(row, b):
            pltpu.make_async_copy(y_h.at[src_v.at[row]], rows_v.at[b],
                                  g_sems[b]).wait()

        def start_scatter(row, b):
            pltpu.async_copy(rows_v.at[b], seg_sh.at[dst_v.at[row]],
                             s_sems[b], add=True)
            pltpu.async_copy(ones_v, cnt_sh.at[dst_v.at[row]],
                             s_sems[b], add=True)

        def wait_scatter(row, b):
            pltpu.make_async_copy(rows_v.at[b], seg_sh.at[dst_v.at[row]],
                                  s_sems[b]).wait()
            pltpu.make_async_copy(ones_v, cnt_sh.at[dst_v.at[row]],
                                  s_sems[b]).wait()

        for b in range(AHEAD):
            start_gather(b, b)

        def step(jn, _):
            j0 = jn * NBUF
            for b in range(NBUF):
                j = j0 + b
                live = j < nrows

                @pl.when(live)
                def _():
                    wait_gather(j, b)
                    start_scatter(j, b)

                bn = (b + AHEAD) % NBUF

                @pl.when((j >= NBUF - AHEAD)
                         & (j - (NBUF - AHEAD) < nrows))
                def _():
                    wait_scatter(j - (NBUF - AHEAD), bn)

                @pl.when(j + AHEAD < nrows)
                def _():
                    start_gather(j + AHEAD, bn)
            return _

        lax.fori_loop(0, ROWS_LOOP // NBUF, step, None)
        for j in range(ROWS_LOOP - (NBUF - AHEAD), ROWS_LOOP):
            @pl.when(j < nrows)
            def _():
                wait_scatter(j, j % NBUF)
        plsc.subcore_barrier()
        pltpu.sync_copy(seg_sh.at[my_acc], seg_o.at[cid, my_acc])
        pltpu.sync_copy(cnt_sh.at[my_acc], cnt_o.at[cid, my_acc])

    run_rel(ed_h, yd_h, segd_o, cntd_o)
    run_rel(es_h, ys_h, segs_o, cnts_o)
    run_rel(ep_h, yp_h, segp_o, cntp_o)


def _sc_scatter(yd, ys, yp, ed, es, ep):
    mesh = plsc.VectorSubcoreMesh(core_axis_name="c", subcore_axis_name="s")
    f32 = jnp.float32
    ones = jnp.ones((CHUNK, CW), f32)
    zseg = jnp.zeros((SEG_PER_TILE, H), f32)
    zcnt = jnp.zeros((SEG_PER_TILE, CW), f32)
    call = pl.kernel(
        _sc_body,
        out_type=[jax.ShapeDtypeStruct((NC, ACC_ROWS, H), f32)] * 3
                 + [jax.ShapeDtypeStruct((NC, ACC_ROWS, CW), f32)] * 3,
        mesh=mesh,
        compiler_params=pltpu.CompilerParams(use_tc_tiling_on_sc=False),
        scratch_types=[
            pltpu.VMEM((ROWS_MAX, CHUNK), jnp.int32),
            pltpu.VMEM((ROWS_MAX, CHUNK), jnp.int32),
            pltpu.VMEM((NBUF, CHUNK, H), f32),
            pltpu.VMEM((CHUNK, CW), f32),
            pltpu.VMEM_SHARED((ACC_ROWS, H), f32),
            pltpu.VMEM_SHARED((ACC_ROWS, CW), f32),
        ] + [pltpu.SemaphoreType.DMA] * (2 * NBUF),
    )
    return call(yd, ys, yp, ed, es, ep, ones, zseg, zcnt)


def _comb_body(segd, segs, segp, cntd, cnts, cntp, ru, rp, uo, po):
    def mean(seg_ref, cnt_ref):
        s = seg_ref[0] + seg_ref[1]
        c = cnt_ref[0][:, 0:1] + cnt_ref[1][:, 0:1]
        return s / jnp.maximum(c, 1.0)

    uo[...] = jnp.maximum(
        mean(segd, cntd) + 0.75 * mean(segs, cnts) + ru[...], 0.0)
    po[...] = jnp.maximum(mean(segp, cntp) + rp[...], 0.0)


def _combine(segd, segs, segp, cntd, cnts, cntp, ru, rp):
    n_blk = N_NODE // BLK
    seg_spec = pl.BlockSpec((NC, BLK, H), lambda i: (0, i, 0))
    cnt_spec = pl.BlockSpec((NC, BLK, CW), lambda i: (0, i, 0))
    r_spec = pl.BlockSpec((BLK, H), lambda i: (i, 0))
    out_shape = jax.ShapeDtypeStruct((N_NODE, H), jnp.float32)
    return pl.pallas_call(
        _comb_body,
        grid=(n_blk,),
        in_specs=[seg_spec] * 3 + [cnt_spec] * 3 + [r_spec] * 2,
        out_specs=[r_spec] * 2,
        out_shape=[out_shape] * 2,
    )(segd, segs, segp, cntd, cnts, cntp, ru, rp)


def kernel(user_x, post_x, edge_index_rev_engages, edge_index_social,
           edge_index_engages, Wl_d, bl_d, Wr_d, Wl_s, bl_s, Wr_s,
           Wl_p, bl_p, Wr_p):
    yd, ys, yp, ru, rp = _project(user_x, post_x, Wl_d, Wl_s, Wl_p,
                                  Wr_d, Wr_s, Wr_p, bl_d, bl_s, bl_p)

    ed = edge_index_rev_engages.reshape(2, N_ROWS, CHUNK)
    es = edge_index_social.reshape(2, N_ROWS, CHUNK)
    ep = edge_index_engages.reshape(2, N_ROWS, CHUNK)

    segd, segs, segp, cntd, cnts, cntp = _sc_scatter(yd, ys, yp, ed, es, ep)

    user_out, post_out = _combine(segd, segs, segp, cntd, cnts, cntp, ru, rp)
    return (user_out, post_out)


# speedup vs baseline: 2.9153x; 1.0439x over previous
import functools

import jax
import jax.numpy as jnp
from jax import lax
from jax.experimental import pallas as pl
from jax.experimental.pallas import tpu as pltpu
from jax.experimental.pallas import tpu_sc as plsc

N_NODE = 10000
D_IN = 128
H = 64
E = 320000

NC = 2
NS = 16
TILES = NC * NS
CHUNK = 128
N_ROWS = E // CHUNK
BASE_ROWS = N_ROWS // TILES
EXTRA = N_ROWS - BASE_ROWS * TILES
ROWS_MAX = BASE_ROWS + 1
ROWS_LOOP = 80
ACC_ROWS = 10240
SEG_PER_TILE = ACC_ROWS // NS
CW = 16
NBUF = 5
AHEAD = 3

BLK = 1000


def _proj_body(u_ref, p_ref, wld, wls, wlp, wrd, wrs, wrp, bld, bls, blp,
               yd, ys, yp, ru, rp):
    u = u_ref[...]
    p = p_ref[...]
    f32 = jnp.float32
    yd[...] = jnp.dot(p, wld[...], preferred_element_type=f32)
    ys[...] = jnp.dot(u, wls[...], preferred_element_type=f32)
    yp[...] = jnp.dot(u, wlp[...], preferred_element_type=f32)
    ru[...] = (jnp.dot(u, wrd[...], preferred_element_type=f32)
               + 0.75 * jnp.dot(u, wrs[...], preferred_element_type=f32)
               + bld[...] + 0.75 * bls[...])
    rp[...] = jnp.dot(p, wrp[...], preferred_element_type=f32) + blp[...]


def _project(user_x, post_x, Wl_d, Wl_s, Wl_p, Wr_d, Wr_s, Wr_p, bl_d, bl_s, bl_p):
    n_blk = N_NODE // BLK
    row_spec = pl.BlockSpec((BLK, D_IN), lambda i: (i, 0))
    w_spec = pl.BlockSpec((D_IN, H), lambda i: (0, 0))
    b_spec = pl.BlockSpec((1, H), lambda i: (0, 0))
    out_spec = pl.BlockSpec((BLK, H), lambda i: (i, 0))
    out_shape = jax.ShapeDtypeStruct((N_NODE, H), jnp.float32)
    return pl.pallas_call(
        _proj_body,
        grid=(n_blk,),
        in_specs=[row_spec, row_spec] + [w_spec] * 6 + [b_spec] * 3,
        out_specs=[out_spec] * 5,
        out_shape=[out_shape] * 5,
    )(user_x, post_x, Wl_d, Wl_s, Wl_p, Wr_d, Wr_s, Wr_p,
      bl_d.reshape(1, H), bl_s.reshape(1, H), bl_p.reshape(1, H))


def _sc_body(yd_h, ys_h, yp_h, ed_h, es_h, ep_h,
             ones_h, zseg_h, zcnt_h,
             segd_o, segs_o, segp_o, cntd_o, cnts_o, cntp_o,
             src_v, dst_v, rows_v, ones_v,
             seg_sh, cnt_sh,
             g0, g1, g2, g3, g4, s0, s1, s2, s3, s4):
    cid = lax.axis_index("c")
    sid = lax.axis_index("s")
    tid = sid * NC + cid
    g_sems = (g0, g1, g2, g3, g4)
    s_sems = (s0, s1, s2, s3, s4)

    row0 = tid * BASE_ROWS + jnp.minimum(tid, EXTRA)
    nrows = jnp.where(tid < EXTRA, BASE_ROWS + 1, BASE_ROWS)

    pltpu.sync_copy(ones_h, ones_v)

    my_acc = pl.ds(sid * SEG_PER_TILE, SEG_PER_TILE)

    def run_rel(e_h, y_h, seg_o, cnt_o):
        pltpu.sync_copy(zseg_h, seg_sh.at[my_acc])
        pltpu.sync_copy(zcnt_h, cnt_sh.at[my_acc])
        pltpu.sync_copy(e_h.at[0, pl.ds(row0, BASE_ROWS)],
                        src_v.at[pl.ds(0, BASE_ROWS)])
        pltpu.sync_copy(e_h.at[1, pl.ds(row0, BASE_ROWS)],
                        dst_v.at[pl.ds(0, BASE_ROWS)])

        @pl.when(tid < EXTRA)
        def _():
            pltpu.sync_copy(e_h.at[0, row0 + BASE_ROWS], src_v.at[BASE_ROWS])
            pltpu.sync_copy(e_h.at[1, row0 + BASE_ROWS], dst_v.at[BASE_ROWS])

        plsc.subcore_barrier()

        def start_gather(row, b):
            pltpu.async_copy(y_h.at[src_v.at[row]], rows_v.at[b], g_sems[b])

        def wait_gather(row, b):
            pltpu.make_async_copy(y_h.at[src_v.at[row]], rows_v.at[b],
                                  g_sems[b]).wait()

        def start_scatter(row, b):
            pltpu.async_copy(rows_v.at[b], seg_sh.at[dst_v.at[row]],
                             s_sems[b], add=True)
            pltpu.async_copy(ones_v, cnt_sh.at[dst_v.at[row]],
                             s_sems[b], add=True)

        def wait_scatter(row, b):
            pltpu.make_async_copy(rows_v.at[b], seg_sh.at[dst_v.at[row]],
                                  s_sems[b]).wait()
            pltpu.make_async_copy(ones_v, cnt_sh.at[dst_v.at[row]],
                                  s_sems[b]).wait()

        for b in range(AHEAD):
            start_gather(b, b)

        def step(jn, _):
            j0 = jn * NBUF
            for b in range(NBUF):
                j = j0 + b
                live = j < nrows

                @pl.when(live)
                def _():
                    wait_gather(j, b)
                    start_scatter(j, b)

                bn = (b + AHEAD) % NBUF

                @pl.when((j >= NBUF - AHEAD)
                         & (j - (NBUF - AHEAD) < nrows))
                def _():
                    wait_scatter(j - (NBUF - AHEAD), bn)

                @pl.when(j + AHEAD < nrows)
                def _():
                    start_gather(j + AHEAD, bn)
            return _

        lax.fori_loop(0, ROWS_LOOP // NBUF, step, None)
        for j in range(ROWS_LOOP - (NBUF - AHEAD), ROWS_LOOP):
            @pl.when(j < nrows)
            def _():
                wait_scatter(j, j % NBUF)
        plsc.subcore_barrier()
        pltpu.sync_copy(seg_sh.at[my_acc], seg_o.at[cid, my_acc])
        pltpu.sync_copy(cnt_sh.at[my_acc], cnt_o.at[cid, my_acc])

    run_rel(ed_h, yd_h, segd_o, cntd_o)
    run_rel(es_h, ys_h, segs_o, cnts_o)
    run_rel(ep_h, yp_h, segp_o, cntp_o)


def _sc_scatter(yd, ys, yp, ed, es, ep):
    mesh = plsc.VectorSubcoreMesh(core_axis_name="c", subcore_axis_name="s")
    f32 = jnp.float32
    ones = jnp.ones((CHUNK, CW), f32)
    zseg = jnp.zeros((SEG_PER_TILE, H), f32)
    zcnt = jnp.zeros((SEG_PER_TILE, CW), f32)
    call = pl.kernel(
        _sc_body,
        out_type=[jax.ShapeDtypeStruct((NC, ACC_ROWS, H), f32)] * 3
                 + [jax.ShapeDtypeStruct((NC, ACC_ROWS, CW), f32)] * 3,
        mesh=mesh,
        compiler_params=pltpu.CompilerParams(use_tc_tiling_on_sc=False),
        scratch_types=[
            pltpu.VMEM((ROWS_MAX, CHUNK), jnp.int32),
            pltpu.VMEM((ROWS_MAX, CHUNK), jnp.int32),
            pltpu.VMEM((NBUF, CHUNK, H), f32),
            pltpu.VMEM((CHUNK, CW), f32),
            pltpu.VMEM_SHARED((ACC_ROWS, H), f32),
            pltpu.VMEM_SHARED((ACC_ROWS, CW), f32),
        ] + [pltpu.SemaphoreType.DMA] * (2 * NBUF),
    )
    return call(yd, ys, yp, ed, es, ep, ones, zseg, zcnt)


def _comb_body(segd, segs, segp, cntd, cnts, cntp, ru, rp, uo, po):
    def mean(seg_ref, cnt_ref):
        s = seg_ref[0] + seg_ref[1]
        c = cnt_ref[0][:, 0:1] + cnt_ref[1][:, 0:1]
        return s / jnp.maximum(c, 1.0)

    uo[...] = jnp.maximum(
        mean(segd, cntd) + 0.75 * mean(segs, cnts) + ru[...], 0.0)
    po[...] = jnp.maximum(mean(segp, cntp) + rp[...], 0.0)


def _combine(segd, segs, segp, cntd, cnts, cntp, ru, rp):
    n_blk = N_NODE // BLK
    seg_spec = pl.BlockSpec((NC, BLK, H), lambda i: (0, i, 0))
    cnt_spec = pl.BlockSpec((NC, BLK, CW), lambda i: (0, i, 0))
    r_spec = pl.BlockSpec((BLK, H), lambda i: (i, 0))
    out_shape = jax.ShapeDtypeStruct((N_NODE, H), jnp.float32)
    return pl.pallas_call(
        _comb_body,
        grid=(n_blk,),
        in_specs=[seg_spec] * 3 + [cnt_spec] * 3 + [r_spec] * 2,
        out_specs=[r_spec] * 2,
        out_shape=[out_shape] * 2,
    )(segd, segs, segp, cntd, cnts, cntp, ru, rp)


def kernel(user_x, post_x, edge_index_rev_engages, edge_index_social,
           edge_index_engages, Wl_d, bl_d, Wr_d, Wl_s, bl_s, Wr_s,
           Wl_p, bl_p, Wr_p):
    yd, ys, yp, ru, rp = _project(user_x, post_x, Wl_d, Wl_s, Wl_p,
                                  Wr_d, Wr_s, Wr_p, bl_d, bl_s, bl_p)

    ed = edge_index_rev_engages.reshape(2, N_ROWS, CHUNK)
    es = edge_index_social.reshape(2, N_ROWS, CHUNK)
    ep = edge_index_engages.reshape(2, N_ROWS, CHUNK)

    segd, segs, segp, cntd, cnts, cntp = _sc_scatter(yd, ys, yp, ed, es, ep)

    user_out, post_out = _combine(segd, segs, segp, cntd, cnts, cntp, ru, rp)
    return (user_out, post_out)
